# Initial kernel scaffold; baseline (speedup 1.0000x reference)
import jax
import jax.numpy as jnp
from jax.experimental import pallas as pl


def kernel(verts_src, trg, edge_len, faces):
    raise NotImplementedError("write your pallas kernel here")


# speedup vs baseline: 1.0173x; 1.0173x over previous
import functools

import jax
import jax.numpy as jnp
import numpy as np
from jax.experimental import pallas as pl
from jax.experimental.pallas import tpu as pltpu

V = 40000
F = 80000
B = 1
S = 5000
SP = 5120
BI = 1024


def _chamfer_body(s_ref, tT_ref, rm_ref, cm_ref):
    i = pl.program_id(0)
    s = s_ref[...]
    tT = tT_ref[...]
    d = jnp.zeros((BI, SP), jnp.float32)
    for c in range(3):
        diff = s[:, c:c + 1] - tT[c:c + 1, :]
        d = d + diff * diff
    rm = jnp.min(d, axis=1)
    rm_ref[...] = rm.reshape(BI // 128, 128)
    cm = jnp.min(d, axis=0)
    cm = cm.reshape(SP // 128, 128)

    @pl.when(i == 0)
    def _():
        cm_ref[...] = cm

    @pl.when(i != 0)
    def _():
        cm_ref[...] = jnp.minimum(cm_ref[...], cm)


def _chamfer(sample_pts, trg_pts):
    big_s = 1e9
    big_t = -1e9
    s = jnp.full((SP, 8), big_s, jnp.float32).at[:S, :3].set(sample_pts)
    t = jnp.full((SP, 8), big_t, jnp.float32).at[:S, :3].set(trg_pts)
    tT = t.T.reshape(8, SP)

    rm, cm = pl.pallas_call(
        _chamfer_body,
        grid=(SP // BI,),
        in_specs=[
            pl.BlockSpec((BI, 8), lambda i: (i, 0)),
            pl.BlockSpec((8, SP), lambda i: (0, 0)),
        ],
        out_specs=[
            pl.BlockSpec((BI // 128, 128), lambda i: (i, 0)),
            pl.BlockSpec((SP // 128, 128), lambda i: (0, 0)),
        ],
        out_shape=[
            jax.ShapeDtypeStruct((SP // 128, 128), jnp.float32),
            jax.ShapeDtypeStruct((SP // 128, 128), jnp.float32),
        ],
        compiler_params=pltpu.CompilerParams(
            dimension_semantics=("arbitrary",)),
    )(s, tT)
    return rm.reshape(SP), cm.reshape(SP)


def _cot(a, b, c):
    e1 = b - a
    e2 = c - a
    cosang = jnp.sum(e1 * e2, axis=-1)
    sinang = jnp.linalg.norm(jnp.cross(e1, e2), axis=-1)
    return cosang / (sinang + 1e-12)


def kernel(verts_src, trg, edge_len, faces):
    f0, f1, f2 = faces[:, 0], faces[:, 1], faces[:, 2]
    v0 = verts_src[f0]
    v1 = verts_src[f1]
    v2 = verts_src[f2]
    cross = jnp.cross(v1 - v0, v2 - v0)
    areas = 0.5 * jnp.linalg.norm(cross, axis=-1)
    logits = jnp.log(areas / jnp.sum(areas) + 1e-12)
    skey = jax.random.key(42)
    ks1, ks2, ks3 = jax.random.split(skey, 3)
    face_idx = jax.random.categorical(ks1, logits, shape=(B, S))
    u = jax.random.uniform(ks2, (B, S, 1), dtype=jnp.float32)
    vv = jax.random.uniform(ks3, (B, S, 1), dtype=jnp.float32)
    su = jnp.sqrt(u)
    w0 = 1.0 - su
    w1 = su * (1.0 - vv)
    w2 = su * vv
    sample_scr = w0 * v0[face_idx] + w1 * v1[face_idx] + w2 * v2[face_idx]
    rm, cm = _chamfer(sample_scr[0], trg[0])
    loss_p0 = jnp.mean(rm[:S]) + jnp.mean(cm[:S])
    loss_n1 = jnp.asarray(1e-5, dtype=jnp.float32)
    c0 = _cot(v0, v1, v2)
    c1 = _cot(v1, v2, v0)
    c2 = _cot(v2, v0, v1)
    src = jnp.concatenate([f1, f2, f2, f0, f0, f1])
    dst = jnp.concatenate([f2, f1, f0, f2, f1, f0])
    w = jnp.concatenate([c0, c0, c1, c1, c2, c2])
    Lv = jax.ops.segment_sum(w[:, None] * verts_src[dst], src, num_segments=V)
    wsum = jax.ops.segment_sum(w, src, num_segments=V)
    safe = jnp.where(wsum > 0, wsum, 1.0)
    norm_w = jnp.where(wsum > 0, 1.0 / safe, 0.0)
    lap = Lv * norm_w[:, None] - verts_src
    loss_laplacian = jnp.mean(jnp.linalg.norm(lap, axis=1))
    e01 = jnp.linalg.norm(v0 - v1, axis=-1)
    e12 = jnp.linalg.norm(v1 - v2, axis=-1)
    e02 = jnp.linalg.norm(v0 - v2, axis=-1)
    elen = jnp.concatenate([e01, e12, e02])
    loss_edge = jnp.mean((elen - edge_len) ** 2)
    return jnp.stack([loss_p0, loss_n1, loss_laplacian, loss_edge])
